# baseline (speedup 1.0000x reference)
import functools

import jax
import jax.numpy as jnp
from jax import lax
from jax.experimental import pallas as pl
from jax.experimental.pallas import tpu as pltpu
from jax.experimental.pallas import tpu_sc as plsc

EMBED = 64
PLOT_DIM = 384
HIDDEN = 128
IDX_CHUNK = 128


def _sc_counts():
    try:
        info = plsc.get_sparse_core_info()
        return int(info.num_cores), int(info.num_subcores)
    except Exception:
        return 2, 16


def _make_gather(batch):
    NC, NS = _sc_counts()
    NW = NC * NS
    b_per_w = batch // NW
    n_chunks = b_per_w // IDX_CHUNK
    assert b_per_w % IDX_CHUNK == 0
    rows_per_w = b_per_w // IDX_CHUNK

    mesh = plsc.VectorSubcoreMesh(core_axis_name="c", subcore_axis_name="s")

    @functools.partial(
        pl.kernel,
        out_type=jax.ShapeDtypeStruct((batch, 2 * EMBED), jnp.float32),
        mesh=mesh,
        scratch_types=[
            pltpu.VMEM((rows_per_w, IDX_CHUNK), jnp.int32),
            pltpu.VMEM((b_per_w, 2 * EMBED), jnp.float32),
            pltpu.SemaphoreType.DMA,
        ],
    )
    def gather1(idx_hbm, table_hbm, out_hbm, idx_v, rows_v, sem):
        wid = lax.axis_index("s") * NC + lax.axis_index("c")
        base = wid * b_per_w
        row0 = wid * rows_per_w
        pltpu.sync_copy(idx_hbm.at[pl.ds(row0, rows_per_w)], idx_v)
        copies = []
        for j in range(n_chunks):
            dst = rows_v.at[pl.ds(j * IDX_CHUNK, IDX_CHUNK)]
            copies.append(
                pltpu.async_copy(table_hbm.at[idx_v.at[j]], dst, sem))
        for c in copies:
            c.wait()
        pltpu.sync_copy(rows_v, out_hbm.at[pl.ds(base, b_per_w)])

    return gather1


def _mlp_body(ue_ref, me_ref, up_ref, mp_ref, plot_ref, w1_ref, b1_ref,
              w2r_ref, b2_ref, out_ref):
    up = up_ref[...]
    mp = mp_ref[...]
    ue = ue_ref[:, 0:EMBED] * (1.0 - up) + ue_ref[:, EMBED:2 * EMBED] * up
    me = me_ref[:, 0:EMBED] * (1.0 - mp) + me_ref[:, EMBED:2 * EMBED] * mp
    x = jnp.dot(ue, w1_ref[0:EMBED, :], preferred_element_type=jnp.float32)
    x += jnp.dot(me, w1_ref[EMBED:2 * EMBED, :],
                 preferred_element_type=jnp.float32)
    x += jnp.dot(plot_ref[...], w1_ref[2 * EMBED:, :],
                 preferred_element_type=jnp.float32)
    x = jnp.maximum(x + b1_ref[...], 0.0)
    out_ref[...] = (jnp.sum(x * w2r_ref[...], axis=1, keepdims=True)
                    + b2_ref[...])


def _make_mlp(batch, blk):
    grid = batch // blk
    in_dim = 2 * EMBED + PLOT_DIM
    return pl.pallas_call(
        _mlp_body,
        grid=(grid,),
        in_specs=[
            pl.BlockSpec((blk, 2 * EMBED), lambda i: (i, 0)),
            pl.BlockSpec((blk, 2 * EMBED), lambda i: (i, 0)),
            pl.BlockSpec((blk, 1), lambda i: (i, 0)),
            pl.BlockSpec((blk, 1), lambda i: (i, 0)),
            pl.BlockSpec((blk, PLOT_DIM), lambda i: (i, 0)),
            pl.BlockSpec((in_dim, HIDDEN), lambda i: (0, 0)),
            pl.BlockSpec((1, HIDDEN), lambda i: (0, 0)),
            pl.BlockSpec((1, HIDDEN), lambda i: (0, 0)),
            pl.BlockSpec((1, 1), lambda i: (0, 0)),
        ],
        out_specs=pl.BlockSpec((blk, 1), lambda i: (i, 0)),
        out_shape=jax.ShapeDtypeStruct((batch, 1), jnp.float32),
    )


@jax.jit
def kernel(users, movies, plot_embeddings, user_table, movie_table,
           W1, b1, W2, b2):
    batch = users.shape[0]
    users = users.astype(jnp.int32)
    movies = movies.astype(jnp.int32)
    upair = (users >> 1).reshape(-1, IDX_CHUNK)
    mpair = (movies >> 1).reshape(-1, IDX_CHUNK)
    uparity = (users & 1).astype(jnp.float32).reshape(batch, 1)
    mparity = (movies & 1).astype(jnp.float32).reshape(batch, 1)
    ut2 = user_table.reshape(-1, 2 * EMBED)
    mt2 = movie_table.reshape(-1, 2 * EMBED)
    gather = _make_gather(batch)
    ue = gather(upair, ut2)
    me = gather(mpair, mt2)
    mlp = _make_mlp(batch, 2048)
    return mlp(ue, me, uparity, mparity, plot_embeddings,
               W1, b1.reshape(1, HIDDEN), W2.reshape(1, HIDDEN),
               b2.reshape(1, 1))


# speedup vs baseline: 2.7276x; 2.7276x over previous
import functools

import jax
import jax.numpy as jnp
from jax import lax
from jax.experimental import pallas as pl
from jax.experimental.pallas import tpu as pltpu
from jax.experimental.pallas import tpu_sc as plsc

EMBED = 64
PLOT_DIM = 384
HIDDEN = 128


def _sc_counts():
    try:
        info = plsc.get_sparse_core_info()
        return int(info.num_cores), int(info.num_subcores)
    except Exception:
        return 2, 16


def _make_gather(batch):
    NC, NS = _sc_counts()
    NW = NC * NS
    b_per_w = batch // NW

    mesh = plsc.VectorSubcoreMesh(core_axis_name="c", subcore_axis_name="s")

    @functools.partial(
        pl.kernel,
        out_type=jax.ShapeDtypeStruct((EMBED, batch), jnp.float32),
        mesh=mesh,
        compiler_params=pltpu.CompilerParams(needs_layout_passes=False),
        scratch_types=[
            pltpu.VMEM((b_per_w,), jnp.int32),
            pltpu.VMEM((8, EMBED, 128), jnp.float32),
            pltpu.VMEM((EMBED, 128), jnp.float32),
            pltpu.SemaphoreType.DMA,
            pltpu.SemaphoreType.DMA,
            pltpu.SemaphoreType.DMA,
            pltpu.SemaphoreType.DMA,
            pltpu.SemaphoreType.DMA,
            pltpu.SemaphoreType.DMA,
            pltpu.SemaphoreType.DMA,
            pltpu.SemaphoreType.DMA,
        ],
    )
    def gather1(idx_hbm, tT_hbm, out, idx_v, ring, stage,
                s0, s1, s2, s3, s4, s5, s6, s7):
        wid = lax.axis_index("s") * NC + lax.axis_index("c")
        base = wid * b_per_w
        i16 = lax.iota(jnp.int32, 16)
        sems = (s0, s1, s2, s3, s4, s5, s6, s7)

        pltpu.sync_copy(idx_hbm.at[pl.ds(base, b_per_w)], idx_v)

        def lane(v, j):
            return jnp.squeeze(lax.slice(v, (j,), (j + 1,)))

        def fetch(i, slot):
            q = pl.multiple_of((i >> 7) << 7, 128)
            pltpu.make_async_copy(
                tT_hbm.at[:, pl.ds(q, 128)], ring.at[slot],
                sems[slot]).start()

        v0 = idx_v[pl.ds(0, 16)]
        for j in range(8):
            fetch(lane(v0, j), j)

        @pl.loop(0, b_per_w // 16)
        def _(g):
            v = idx_v[pl.ds(g * 16, 16)]
            for j in range(16):
                r = g * 16 + j
                slot = j % 8
                if j == 8:
                    vn_off = jnp.where(g + 1 < b_per_w // 16,
                                       (g + 1) * 16, 0)
                    v = idx_v[pl.ds(vn_off, 16)]
                iv = idx_v[pl.ds(g * 16, 16)]
                i = lane(iv, j)
                pltpu.make_async_copy(
                    tT_hbm.at[:, pl.ds(0, 128)], ring.at[slot],
                    sems[slot]).wait()
                c_vec = jnp.zeros((16,), jnp.int32) + (i & 127)
                col_vec = jnp.zeros((16,), jnp.int32) + (r & 127)
                slot_vec = jnp.full((16,), slot, jnp.int32)
                for q4 in range(EMBED // 16):
                    e_vec = q4 * 16 + i16
                    vals = plsc.load_gather(
                        ring, [slot_vec, e_vec, c_vec])
                    plsc.store_scatter(stage, [e_vec, col_vec], vals)

                nxt = lane(v, j - 8 if j >= 8 else j + 8)

                @pl.when(r + 8 < b_per_w)
                def _():
                    fetch(nxt, slot)

            @pl.when((g & 7) == 7)
            def _():
                off = pl.multiple_of((g >> 3) << 7, 128)
                pltpu.sync_copy(stage, out.at[:, pl.ds(base + off, 128)])

    return gather1


def _mlp_body(ue_ref, me_ref, plot_ref, w1_ref, b1_ref, w2r_ref, b2_ref,
              out_ref):
    dn = (((0,), (0,)), ((), ()))
    x = lax.dot_general(ue_ref[...], w1_ref[0:EMBED, :], dn,
                        preferred_element_type=jnp.float32)
    x += lax.dot_general(me_ref[...], w1_ref[EMBED:2 * EMBED, :], dn,
                         preferred_element_type=jnp.float32)
    x += jnp.dot(plot_ref[...], w1_ref[2 * EMBED:, :],
                 preferred_element_type=jnp.float32)
    x = jnp.maximum(x + b1_ref[...], 0.0)
    out_ref[...] = (jnp.sum(x * w2r_ref[...], axis=1, keepdims=True)
                    + b2_ref[...])


def _make_mlp(batch, blk):
    grid = batch // blk
    in_dim = 2 * EMBED + PLOT_DIM
    return pl.pallas_call(
        _mlp_body,
        grid=(grid,),
        in_specs=[
            pl.BlockSpec((EMBED, blk), lambda i: (0, i)),
            pl.BlockSpec((EMBED, blk), lambda i: (0, i)),
            pl.BlockSpec((blk, PLOT_DIM), lambda i: (i, 0)),
            pl.BlockSpec((in_dim, HIDDEN), lambda i: (0, 0)),
            pl.BlockSpec((1, HIDDEN), lambda i: (0, 0)),
            pl.BlockSpec((1, HIDDEN), lambda i: (0, 0)),
            pl.BlockSpec((1, 1), lambda i: (0, 0)),
        ],
        out_specs=pl.BlockSpec((blk, 1), lambda i: (i, 0)),
        out_shape=jax.ShapeDtypeStruct((batch, 1), jnp.float32),
    )


@jax.jit
def kernel(users, movies, plot_embeddings, user_table, movie_table,
           W1, b1, W2, b2):
    batch = users.shape[0]
    users = users.astype(jnp.int32)
    movies = movies.astype(jnp.int32)
    gather = _make_gather(batch)
    ue = gather(users, user_table.T)
    me = gather(movies, movie_table.T)
    mlp = _make_mlp(batch, 2048)
    return mlp(ue, me, plot_embeddings,
               W1, b1.reshape(1, HIDDEN), W2.reshape(1, HIDDEN),
               b2.reshape(1, 1))
